# Initial kernel scaffold; baseline (speedup 1.0000x reference)
import jax
import jax.numpy as jnp
from jax.experimental import pallas as pl


def kernel(y, x, ray_dir, sh_data):
    raise NotImplementedError("write your pallas kernel here")


# speedup vs baseline: 19.9407x; 19.9407x over previous
import functools
import math

import jax
import jax.numpy as jnp
from jax import lax
from jax.experimental import pallas as pl
from jax.experimental.pallas import tpu as pltpu
from jax.experimental.pallas import tpu_sc as plsc

_SQ2 = math.sqrt(2.0)
_PI4 = 4.0 * math.pi
_N00 = math.sqrt(1.0 / _PI4)
_N10 = math.sqrt(3.0 / _PI4)
_N11 = math.sqrt(3.0 / _PI4 / 2.0)
_N20 = math.sqrt(5.0 / _PI4)
_N21 = math.sqrt(5.0 / _PI4 / 6.0)
_N22 = math.sqrt(5.0 / _PI4 / 24.0)
_N30 = math.sqrt(7.0 / _PI4)
_N31 = math.sqrt(7.0 / _PI4 / 12.0)
_N32 = math.sqrt(7.0 / _PI4 / 120.0)
_N33 = math.sqrt(7.0 / _PI4 / 720.0)

_LANES = 16
_IDXW = 128
_CHUNK = 2048


def _sc_gather(table, yf, xf, res_y, res_x):
    nrows, d = table.shape
    b = yf.shape[0]
    info = plsc.get_sparse_core_info()
    nc, ns = info.num_cores, info.num_subcores
    nw = nc * ns
    b_per_w = b // nw
    assert b % (nw * _CHUNK) == 0
    nchunk = b_per_w // _CHUNK
    nsub = _CHUNK // _IDXW

    mesh = plsc.VectorSubcoreMesh(core_axis_name="c", subcore_axis_name="s")

    @functools.partial(
        pl.kernel,
        mesh=mesh,
        compiler_params=pltpu.CompilerParams(use_tc_tiling_on_sc=False),
        out_type=jax.ShapeDtypeStruct((b, d), jnp.float32),
        scratch_types=[
            pltpu.VMEM((_CHUNK,), jnp.float32),
            pltpu.VMEM((_CHUNK,), jnp.float32),
            pltpu.VMEM((nsub, _IDXW), jnp.int32),
            pltpu.VMEM((_CHUNK, d), jnp.float32),
            pltpu.SemaphoreType.DMA,
        ],
    )
    def gather_k(table_hbm, y_hbm, x_hbm, out_hbm, y_v, x_v, idx_v, rows_v, sem):
        wid = lax.axis_index("s") * nc + lax.axis_index("c")
        base = wid * b_per_w
        ymax = float(res_y - 1)
        xmax = float(res_x - 1)
        for c in range(nchunk):
            off = base + c * _CHUNK
            pltpu.sync_copy(y_hbm.at[pl.ds(off, _CHUNK)], y_v)
            pltpu.sync_copy(x_hbm.at[pl.ds(off, _CHUNK)], x_v)
            for k in range(nsub):
                def grp(g, carry, k=k):
                    s = k * _IDXW + g * _LANES
                    yv = y_v[pl.ds(s, _LANES)]
                    xv = x_v[pl.ds(s, _LANES)]
                    yc = jnp.minimum(jnp.maximum(yv, 0.0), ymax)
                    xc = jnp.minimum(jnp.maximum(xv, 0.0), xmax)
                    idx_v[k, pl.ds(g * _LANES, _LANES)] = (
                        yc * float(res_x) + xc).astype(jnp.int32)
                    return carry
                lax.fori_loop(0, _IDXW // _LANES, grp, 0)
            copies = [
                pltpu.async_copy(
                    table_hbm.at[idx_v.at[k]],
                    rows_v.at[pl.ds(k * _IDXW, _IDXW)],
                    sem,
                )
                for k in range(nsub)
            ]
            for cp in copies:
                cp.wait()
            pltpu.sync_copy(rows_v, out_hbm.at[pl.ds(off, _CHUNK)])

    return gather_k(table, yf, xf)


def _tc_body(g_ref, x_ref, y_ref, z_ref, o_ref):
    x = x_ref[...]
    y = y_ref[...]
    z = z_ref[...]
    s = x * x + y * y + z * z
    r = jnp.sqrt(s) + 1e-14
    rinv = 1.0 / r
    ct = jnp.clip(z * rinv, -1.0, 1.0)
    st = jnp.sqrt(jnp.maximum(1.0 - ct * ct, 1e-14))
    rhoinv = lax.rsqrt(jnp.maximum(x * x + y * y, 1e-30))
    cp = x * rhoinv
    sp = y * rhoinv
    c2 = cp * cp - sp * sp
    s2 = 2.0 * cp * sp
    c3 = cp * c2 - sp * s2
    s3 = sp * c2 + cp * s2
    st2 = st * st
    p21 = 3.0 * ct * st
    p22 = 3.0 * st2
    p31 = 1.5 * st * (5.0 * ct * ct - 1.0)
    p32 = 15.0 * ct * st2
    p33 = 15.0 * st2 * st
    cols = (
        jnp.full_like(ct, _N00),
        (-_SQ2 * _N11) * st * sp,
        _N10 * ct,
        (-_SQ2 * _N11) * st * cp,
        (_SQ2 * _N22) * p22 * s2,
        (-_SQ2 * _N21) * p21 * sp,
        _N20 * (1.5 * ct * ct - 0.5),
        (-_SQ2 * _N21) * p21 * cp,
        (_SQ2 * _N22) * p22 * c2,
        (-_SQ2 * _N33) * p33 * s3,
        (_SQ2 * _N32) * p32 * s2,
        (-_SQ2 * _N31) * p31 * sp,
        _N30 * ((2.5 * ct * ct - 1.5) * ct),
        (-_SQ2 * _N31) * p31 * cp,
        (_SQ2 * _N32) * p32 * c2,
        (-_SQ2 * _N33) * p33 * c3,
    )
    for j in range(3):
        acc = cols[0] * g_ref[j]
        for i in range(1, 16):
            acc = acc + cols[i] * g_ref[3 * i + j]
        o_ref[j] = jnp.clip(acc, 0.0, 1.0)


def _tc_contract(g_t, xs, ys, zs):
    nk = g_t.shape[0]
    nr = g_t.shape[1]
    rblk = 64
    grid = (nr // rblk,)
    return pl.pallas_call(
        _tc_body,
        grid=grid,
        in_specs=[
            pl.BlockSpec((nk, rblk, 128), lambda i: (0, i, 0)),
            pl.BlockSpec((rblk, 128), lambda i: (i, 0)),
            pl.BlockSpec((rblk, 128), lambda i: (i, 0)),
            pl.BlockSpec((rblk, 128), lambda i: (i, 0)),
        ],
        out_specs=pl.BlockSpec((3, rblk, 128), lambda i: (0, i, 0)),
        out_shape=jax.ShapeDtypeStruct((3, nr, 128), jnp.float32),
    )(g_t, xs, ys, zs)


def kernel(y, x, ray_dir, sh_data):
    res_y, res_x, nco, nch = sh_data.shape
    d = nco * nch
    b = y.shape[0]
    nr = b // 128
    table = sh_data.reshape(res_y * res_x, d)
    g = _sc_gather(table, y, x, res_y, res_x)
    g_t = g.T.reshape(d, nr, 128)
    xs = ray_dir[:, 0].reshape(nr, 128)
    ys = ray_dir[:, 1].reshape(nr, 128)
    zs = ray_dir[:, 2].reshape(nr, 128)
    out3 = _tc_contract(g_t, xs, ys, zs)
    return out3.reshape(3, b).T
